# Initial kernel scaffold; baseline (speedup 1.0000x reference)
import functools

import jax
import jax.numpy as jnp
from jax import lax
from jax.experimental import pallas as pl
from jax.experimental.pallas import tpu as pltpu
from jax.experimental.pallas import tpu_sc as plsc

MAXLEN = 200
EMBED = 32
BATCH = 4096
ROWS = BATCH * MAXLEN
NC, NS = 2, 16
NW = NC * NS
RPW = ROWS // NW
SUB = 128
NSUB = 25
CHUNK = SUB * NSUB
NCHUNK = RPW // CHUNK
REPS = CHUNK // MAXLEN

_mesh = plsc.VectorSubcoreMesh(core_axis_name="c", subcore_axis_name="s")


@functools.partial(
    pl.kernel,
    mesh=_mesh,
    out_type=jax.ShapeDtypeStruct((ROWS, EMBED), jnp.float32),
    scratch_types=[
        pltpu.VMEM((NSUB, SUB), jnp.int32),
        pltpu.VMEM((CHUNK, EMBED), jnp.float32),
        pltpu.VMEM((MAXLEN, EMBED), jnp.float32),
        pltpu.SemaphoreType.DMA,
    ],
)
def _embed_kernel(idx_hbm, tok_hbm, pos_hbm, out_hbm, idx_v, rows_v, pos_v, sem):
    wid = lax.axis_index("s") * NC + lax.axis_index("c")
    pltpu.sync_copy(pos_hbm, pos_v)

    def chunk_body(g, carry):
        base = wid * RPW + g * CHUNK
        idx_row = wid * (RPW // SUB) + g * NSUB
        pltpu.sync_copy(idx_hbm.at[pl.ds(idx_row, NSUB)], idx_v)
        copies = []
        for j in range(NSUB):
            copies.append(
                pltpu.async_copy(
                    tok_hbm.at[idx_v.at[j]],
                    rows_v.at[pl.ds(j * SUB, SUB)],
                    sem,
                )
            )
        for c in copies:
            c.wait()

        def pos_body(s, c2):
            p0 = pos_v[s, pl.ds(0, 16)]
            p1 = pos_v[s, pl.ds(16, 16)]
            for rep in range(REPS):
                r = rep * MAXLEN + s
                rows_v[r, pl.ds(0, 16)] = rows_v[r, pl.ds(0, 16)] + p0
                rows_v[r, pl.ds(16, 16)] = rows_v[r, pl.ds(16, 16)] + p1
            return c2

        lax.fori_loop(0, MAXLEN, pos_body, 0)
        pltpu.sync_copy(rows_v, out_hbm.at[pl.ds(base, CHUNK)])
        return carry

    lax.fori_loop(0, NCHUNK, chunk_body, 0)


def kernel(x, token_table, pos_table):
    xf = x.astype(jnp.int32).reshape(ROWS // SUB, SUB)
    out = _embed_kernel(xf, token_table, pos_table)
    return out.reshape(BATCH, MAXLEN, EMBED)


# speedup vs baseline: 1.4281x; 1.4281x over previous
import functools

import jax
import jax.numpy as jnp
from jax import lax
from jax.experimental import pallas as pl
from jax.experimental.pallas import tpu as pltpu
from jax.experimental.pallas import tpu_sc as plsc

MAXLEN = 200
EMBED = 32
BATCH = 4096
ROWS = BATCH * MAXLEN
NC, NS = 2, 16
NW = NC * NS
BPW = BATCH // NW
BPC = 8
NCHUNK = BPW // BPC
CHUNK = BPC * MAXLEN
REPS = CHUNK // MAXLEN
SPLIT = 128

_mesh = plsc.VectorSubcoreMesh(core_axis_name="c", subcore_axis_name="s")


@functools.partial(
    pl.kernel,
    mesh=_mesh,
    out_type=jax.ShapeDtypeStruct((ROWS, EMBED), jnp.float32),
    compiler_params=pltpu.CompilerParams(use_tc_tiling_on_sc=False),
    scratch_types=[
        pltpu.VMEM((BPC, MAXLEN), jnp.int32),
        pltpu.VMEM((CHUNK, EMBED), jnp.float32),
        pltpu.VMEM((MAXLEN, EMBED), jnp.float32),
        pltpu.SemaphoreType.DMA,
    ],
)
def _embed_kernel(idx_hbm, tok_hbm, pos_hbm, out_hbm, idx_v, rows_v, pos_v, sem):
    wid = lax.axis_index("s") * NC + lax.axis_index("c")
    pltpu.sync_copy(pos_hbm, pos_v)

    def chunk_body(g, carry):
        b0 = wid * BPW + g * BPC
        pltpu.sync_copy(idx_hbm.at[pl.ds(b0, BPC)], idx_v)
        copies = []
        for j in range(BPC):
            copies.append(
                pltpu.async_copy(
                    tok_hbm.at[idx_v.at[j, pl.ds(0, SPLIT)]],
                    rows_v.at[pl.ds(j * MAXLEN, SPLIT)],
                    sem,
                )
            )
            copies.append(
                pltpu.async_copy(
                    tok_hbm.at[idx_v.at[j, pl.ds(SPLIT, MAXLEN - SPLIT)]],
                    rows_v.at[pl.ds(j * MAXLEN + SPLIT, MAXLEN - SPLIT)],
                    sem,
                )
            )
        for c in copies:
            c.wait()

        def pos_body(s, c2):
            p0 = pos_v[s, pl.ds(0, 16)]
            p1 = pos_v[s, pl.ds(16, 16)]
            for rep in range(REPS):
                r = rep * MAXLEN + s
                rows_v[r, pl.ds(0, 16)] = rows_v[r, pl.ds(0, 16)] + p0
                rows_v[r, pl.ds(16, 16)] = rows_v[r, pl.ds(16, 16)] + p1
            return c2

        lax.fori_loop(0, MAXLEN, pos_body, 0)
        pltpu.sync_copy(rows_v, out_hbm.at[pl.ds(b0 * MAXLEN, CHUNK)])
        return carry

    lax.fori_loop(0, NCHUNK, chunk_body, 0)


def kernel(x, token_table, pos_table):
    out = _embed_kernel(x.astype(jnp.int32), token_table, pos_table)
    return out.reshape(BATCH, MAXLEN, EMBED)


# speedup vs baseline: 1.4375x; 1.0066x over previous
import functools

import jax
import jax.numpy as jnp
from jax import lax
from jax.experimental import pallas as pl
from jax.experimental.pallas import tpu as pltpu
from jax.experimental.pallas import tpu_sc as plsc

MAXLEN = 200
EMBED = 32
BATCH = 4096
ROWS = BATCH * MAXLEN
NC, NS = 2, 16
NW = NC * NS
BPW = BATCH // NW
RPW = BPW * MAXLEN
BPC = 8
NCHUNK = BPW // BPC
CHUNK = BPC * MAXLEN
SPLIT = 128

_mesh = plsc.VectorSubcoreMesh(core_axis_name="c", subcore_axis_name="s")


@functools.partial(
    pl.kernel,
    mesh=_mesh,
    out_type=jax.ShapeDtypeStruct((BATCH, MAXLEN, EMBED), jnp.float32),
    compiler_params=pltpu.CompilerParams(use_tc_tiling_on_sc=False),
    scratch_types=[
        pltpu.VMEM((RPW,), jnp.int32),
        pltpu.VMEM((BPC, MAXLEN, EMBED), jnp.float32),
        pltpu.VMEM((MAXLEN, EMBED), jnp.float32),
        pltpu.SemaphoreType.DMA,
    ],
)
def _embed_kernel(idx_hbm, tok_hbm, pos_hbm, out_hbm, idx_v, rows_v, pos_v, sem):
    wid = lax.axis_index("s") * NC + lax.axis_index("c")
    pltpu.sync_copy(pos_hbm, pos_v)
    pltpu.sync_copy(idx_hbm.at[pl.ds(wid * RPW, RPW)], idx_v)

    def chunk_body(g, carry):
        copies = []
        for jb in range(BPC):
            q = g * CHUNK + jb * MAXLEN
            copies.append(
                pltpu.async_copy(
                    tok_hbm.at[idx_v.at[pl.ds(q, SPLIT)]],
                    rows_v.at[jb, pl.ds(0, SPLIT)],
                    sem,
                )
            )
            copies.append(
                pltpu.async_copy(
                    tok_hbm.at[idx_v.at[pl.ds(q + SPLIT, MAXLEN - SPLIT)]],
                    rows_v.at[jb, pl.ds(SPLIT, MAXLEN - SPLIT)],
                    sem,
                )
            )
        for c in copies:
            c.wait()

        def pos_body(s, c2):
            p0 = pos_v[s, pl.ds(0, 16)]
            p1 = pos_v[s, pl.ds(16, 16)]
            for jb in range(BPC):
                rows_v[jb, s, pl.ds(0, 16)] = rows_v[jb, s, pl.ds(0, 16)] + p0
                rows_v[jb, s, pl.ds(16, 16)] = rows_v[jb, s, pl.ds(16, 16)] + p1
            return c2

        lax.fori_loop(0, MAXLEN, pos_body, 0)
        b0 = wid * BPW + g * BPC
        pltpu.sync_copy(rows_v, out_hbm.at[pl.ds(b0, BPC)])
        return carry

    lax.fori_loop(0, NCHUNK, chunk_body, 0)


def kernel(x, token_table, pos_table):
    xf = x.astype(jnp.int32).reshape(ROWS)
    return _embed_kernel(xf, token_table, pos_table)
